# baseline (speedup 1.0000x reference)
import functools

import jax
import jax.numpy as jnp
from jax import lax
from jax.experimental import pallas as pl
from jax.experimental.pallas import tpu as pltpu
from jax.experimental.pallas import tpu_sc as plsc

B = 4
T = 8192
D = 128
N = B * T
NC = 2
NS = 16
NW = NC * NS
RPW = N // NW
G = 128
NG = RPW // G
NBUF = 6


def _body(idx_hbm, tok_hbm, pos_hbm, out_hbm, idx_v,
          buf0, buf1, buf2, buf3, buf4, buf5, semp, sema, sems):
    bufs = [buf0, buf1, buf2, buf3, buf4, buf5]
    cid = lax.axis_index("c")
    sid = lax.axis_index("s")
    j = cid * (NW // (NC * B)) + sid // B
    b = lax.rem(sid, B)
    row_base = pl.multiple_of(b * T + j * RPW, RPW)
    pos_base = pl.multiple_of(j * RPW, RPW)

    pltpu.sync_copy(idx_hbm.at[pl.ds(pl.multiple_of(row_base // G, NG), NG)],
                    idx_v)

    def start_pos(g):
        return pltpu.async_copy(
            pos_hbm.at[pl.ds(pos_base + g * G, G)], bufs[g % NBUF],
            semp.at[g % NBUF])

    def start_gather_add(g):
        return pltpu.async_copy(
            tok_hbm.at[idx_v.at[g]], bufs[g % NBUF], sema.at[g % NBUF],
            add=True)

    def start_store(g):
        return pltpu.async_copy(
            bufs[g % NBUF], out_hbm.at[pl.ds(row_base + g * G, G)],
            sems.at[g % NBUF])

    cp_p = [None] * NG
    cp_a = [None] * NG
    cp_s = [None] * NG

    for g in range(min(NBUF, NG)):
        cp_p[g] = start_pos(g)

    for g in range(NG):
        cp_p[g].wait()
        cp_a[g] = start_gather_add(g)
        if g >= 2:
            cp_a[g - 2].wait()
            cp_s[g - 2] = start_store(g - 2)
        if 3 <= g and g + NBUF - 3 < NG:
            cp_s[g - 3].wait()
            cp_p[g + NBUF - 3] = start_pos(g + NBUF - 3)

    for g in range(max(0, NG - 2), NG):
        cp_a[g].wait()
        cp_s[g] = start_store(g)
    for g in range(max(0, NG - 3), NG):
        cp_s[g].wait()


@jax.jit
def _run(idx2d, token_table, pos_table):
    mesh = plsc.VectorSubcoreMesh(core_axis_name="c", subcore_axis_name="s")
    kfn = functools.partial(
        pl.kernel,
        mesh=mesh,
        out_type=jax.ShapeDtypeStruct((N, D), jnp.float32),
        scratch_types=[
            pltpu.VMEM((NG, G), jnp.int32),
            pltpu.VMEM((G, D), jnp.float32),
            pltpu.VMEM((G, D), jnp.float32),
            pltpu.VMEM((G, D), jnp.float32),
            pltpu.VMEM((G, D), jnp.float32),
            pltpu.VMEM((G, D), jnp.float32),
            pltpu.VMEM((G, D), jnp.float32),
            pltpu.SemaphoreType.DMA((NBUF,)),
            pltpu.SemaphoreType.DMA((NBUF,)),
            pltpu.SemaphoreType.DMA((NBUF,)),
        ],
    )(_body)
    return kfn(idx2d, token_table, pos_table)


def kernel(idx, token_table, pos_table):
    idx2d = idx.astype(jnp.int32).reshape(N // G, G)
    out = _run(idx2d, token_table, pos_table)
    return out.reshape(B, T, D)


# speedup vs baseline: 1.0474x; 1.0474x over previous
import functools

import jax
import jax.numpy as jnp
from jax import lax
from jax.experimental import pallas as pl
from jax.experimental.pallas import tpu as pltpu
from jax.experimental.pallas import tpu_sc as plsc

B = 4
T = 8192
D = 128
N = B * T
NC = 2
NS = 16
NW = NC * NS
RPW = N // NW
G = 128
NG = RPW // G
NBUF = 5


def _body(idx_hbm, tok_hbm, pos_hbm, out_hbm, idx_v,
          buf0, buf1, buf2, buf3, buf4, pos_sh, semp, sema, sems):
    bufs = [buf0, buf1, buf2, buf3, buf4]
    cid = lax.axis_index("c")
    sid = lax.axis_index("s")
    j = cid * (NW // (NC * B)) + sid // B
    b = lax.rem(sid, B)
    row_base = pl.multiple_of(b * T + j * RPW, RPW)
    sh_base = pl.multiple_of((sid // B) * RPW, RPW)

    stripe = (T // NC) // NS
    pltpu.sync_copy(
        pos_hbm.at[pl.ds(pl.multiple_of(cid * (T // NC) + sid * stripe, stripe),
                         stripe)],
        pos_sh.at[pl.ds(pl.multiple_of(sid * stripe, stripe), stripe)])

    pltpu.sync_copy(idx_hbm.at[pl.ds(pl.multiple_of(row_base // G, NG), NG)],
                    idx_v)
    plsc.subcore_barrier()

    def start_pos(g):
        return pltpu.async_copy(
            pos_sh.at[pl.ds(sh_base + g * G, G)], bufs[g % NBUF],
            semp.at[g % NBUF])

    def start_gather_add(g):
        return pltpu.async_copy(
            tok_hbm.at[idx_v.at[g]], bufs[g % NBUF], sema.at[g % NBUF],
            add=True)

    def start_store(g):
        return pltpu.async_copy(
            bufs[g % NBUF], out_hbm.at[pl.ds(row_base + g * G, G)],
            sems.at[g % NBUF])

    cp_p = [None] * NG
    cp_a = [None] * NG
    cp_s = [None] * NG

    for g in range(min(NBUF, NG)):
        cp_p[g] = start_pos(g)

    for g in range(NG):
        cp_p[g].wait()
        cp_a[g] = start_gather_add(g)
        if g >= 2:
            cp_a[g - 2].wait()
            cp_s[g - 2] = start_store(g - 2)
        if 3 <= g and g + NBUF - 3 < NG:
            cp_s[g - 3].wait()
            cp_p[g + NBUF - 3] = start_pos(g + NBUF - 3)

    for g in range(max(0, NG - 2), NG):
        cp_a[g].wait()
        cp_s[g] = start_store(g)
    for g in range(max(0, NG - 3), NG):
        cp_s[g].wait()


@jax.jit
def _run(idx2d, token_table, pos_table):
    mesh = plsc.VectorSubcoreMesh(core_axis_name="c", subcore_axis_name="s")
    kfn = functools.partial(
        pl.kernel,
        mesh=mesh,
        out_type=jax.ShapeDtypeStruct((N, D), jnp.float32),
        scratch_types=[
            pltpu.VMEM((NG, G), jnp.int32),
            pltpu.VMEM((G, D), jnp.float32),
            pltpu.VMEM((G, D), jnp.float32),
            pltpu.VMEM((G, D), jnp.float32),
            pltpu.VMEM((G, D), jnp.float32),
            pltpu.VMEM((G, D), jnp.float32),
            pltpu.VMEM_SHARED((T // NC, D), jnp.float32),
            pltpu.SemaphoreType.DMA((NBUF,)),
            pltpu.SemaphoreType.DMA((NBUF,)),
            pltpu.SemaphoreType.DMA((NBUF,)),
        ],
    )(_body)
    return kfn(idx2d, token_table, pos_table)


def kernel(idx, token_table, pos_table):
    idx2d = idx.astype(jnp.int32).reshape(N // G, G)
    out = _run(idx2d, token_table, pos_table)
    return out.reshape(B, T, D)


# speedup vs baseline: 1.0582x; 1.0103x over previous
import functools

import jax
import jax.numpy as jnp
from jax import lax
from jax.experimental import pallas as pl
from jax.experimental.pallas import tpu as pltpu
from jax.experimental.pallas import tpu_sc as plsc

B = 4
T = 8192
D = 128
N = B * T
NC = 2
NS = 16
NW = NC * NS
RPW = N // NW
G = 128
NG = RPW // G
NBUF = 5


def _body(idx_hbm, tok_hbm, pos_hbm, out_hbm, idx_v,
          buf0, buf1, buf2, buf3, buf4, pos_sh, semp, sema, sems):
    bufs = [buf0, buf1, buf2, buf3, buf4]
    cid = lax.axis_index("c")
    sid = lax.axis_index("s")
    j = cid * (NW // (NC * B)) + sid // B
    b = lax.rem(sid, B)
    row_base = pl.multiple_of(b * T + j * RPW, RPW)
    sh_base = pl.multiple_of((sid // B) * RPW, RPW)

    stripe = (T // NC) // NS
    pltpu.sync_copy(
        pos_hbm.at[pl.ds(pl.multiple_of(cid * (T // NC) + sid * stripe, stripe),
                         stripe)],
        pos_sh.at[pl.ds(pl.multiple_of(sid * stripe, stripe), stripe)])

    pltpu.sync_copy(
        idx_hbm.at[b, pl.ds(pl.multiple_of(j * RPW, RPW), RPW)],
        idx_v)
    plsc.subcore_barrier()

    def start_pos(g):
        return pltpu.async_copy(
            pos_sh.at[pl.ds(sh_base + g * G, G)], bufs[g % NBUF],
            semp.at[g % NBUF])

    def start_gather_add(g):
        return pltpu.async_copy(
            tok_hbm.at[idx_v.at[pl.ds(g * G, G)]], bufs[g % NBUF],
            sema.at[g % NBUF], add=True)

    def start_store(g):
        return pltpu.async_copy(
            bufs[g % NBUF], out_hbm.at[pl.ds(row_base + g * G, G)],
            sems.at[g % NBUF])

    cp_p = [None] * NG
    cp_a = [None] * NG
    cp_s = [None] * NG

    for g in range(min(NBUF, NG)):
        cp_p[g] = start_pos(g)

    for g in range(NG):
        cp_p[g].wait()
        cp_a[g] = start_gather_add(g)
        if g >= 2:
            cp_a[g - 2].wait()
            cp_s[g - 2] = start_store(g - 2)
        if 3 <= g and g + NBUF - 3 < NG:
            cp_s[g - 3].wait()
            cp_p[g + NBUF - 3] = start_pos(g + NBUF - 3)

    for g in range(max(0, NG - 2), NG):
        cp_a[g].wait()
        cp_s[g] = start_store(g)
    for g in range(max(0, NG - 3), NG):
        cp_s[g].wait()


@jax.jit
def _run(idx2d, token_table, pos_table):
    mesh = plsc.VectorSubcoreMesh(core_axis_name="c", subcore_axis_name="s")
    kfn = functools.partial(
        pl.kernel,
        mesh=mesh,
        out_type=jax.ShapeDtypeStruct((N, D), jnp.float32),
        scratch_types=[
            pltpu.VMEM((RPW,), jnp.int32),
            pltpu.VMEM((G, D), jnp.float32),
            pltpu.VMEM((G, D), jnp.float32),
            pltpu.VMEM((G, D), jnp.float32),
            pltpu.VMEM((G, D), jnp.float32),
            pltpu.VMEM((G, D), jnp.float32),
            pltpu.VMEM_SHARED((T // NC, D), jnp.float32),
            pltpu.SemaphoreType.DMA((NBUF,)),
            pltpu.SemaphoreType.DMA((NBUF,)),
            pltpu.SemaphoreType.DMA((NBUF,)),
        ],
    )(_body)
    return kfn(idx2d, token_table, pos_table)


def kernel(idx, token_table, pos_table):
    out = _run(idx.astype(jnp.int32), token_table, pos_table)
    return out.reshape(B, T, D)
